# Initial kernel scaffold; baseline (speedup 1.0000x reference)
import jax
import jax.numpy as jnp
from jax import lax
from jax.experimental import pallas as pl
from jax.experimental.pallas import tpu as pltpu, tpu_sc as plsc

B, P, L, M, LE = 32, 16, 256, 64, 512
NPATH = B * P
LANES = 16
QV = LE // LANES


def _sc_body(ev_hbm, q_hbm, mu_hbm, al_hbm, be_hbm, out_hbm,
             ev_v, q_v, idx_v, dt_v, row_v, mu_v, al_v, be_v, out_v,
             sem0, sem1, sem2, sem3):
    nc = 2
    wid = lax.axis_index("s") * nc + lax.axis_index("c")
    iota = lax.iota(jnp.int32, LANES)

    def path_body(i, carry):
        path = wid * 16 + i
        b = path // P
        p = path - b * P
        base = b * (M * P) + p

        pltpu.sync_copy(ev_hbm.at[path], ev_v)
        pltpu.sync_copy(q_hbm.at[path], q_v)

        for k in range(M // LANES):
            row_v[pl.ds(k * LANES, LANES)] = base + (k * LANES + iota) * P

        cp0 = pltpu.async_copy(mu_hbm.at[row_v], mu_v, sem0)
        cp1 = pltpu.async_copy(al_hbm.at[row_v], al_v, sem1)
        cp2 = pltpu.async_copy(be_hbm.at[row_v], be_v, sem2)

        def bs_body(lev, bcarry):
            q = q_v[pl.ds(lev * LANES, LANES)]
            lo = jnp.zeros((LANES,), jnp.int32)
            hi = jnp.full((LANES,), L, jnp.int32)
            for _step in range(8):
                mid = lax.shift_right_logical(lo + hi, 1)
                ev_mid = plsc.load_gather(ev_v, [mid])
                pred = ev_mid < q
                lo = jnp.where(pred, mid + 1, lo)
                hi = jnp.where(pred, hi, mid)
            last = lo - 1
            clamped = jnp.maximum(last, 0)
            t_last = plsc.load_gather(ev_v, [clamped])
            t_last = jnp.where(last < 0, jnp.zeros_like(t_last), t_last)
            idx_v[pl.ds(lev * LANES, LANES)] = clamped
            dt_v[pl.ds(lev * LANES, LANES)] = q - t_last
            return bcarry

        lax.fori_loop(0, QV, bs_body, 0)

        cp0.wait()
        cp1.wait()
        cp2.wait()

        def q_body(lev, qcarry):
            idxv = idx_v[pl.ds(lev * LANES, LANES)]
            dtv = dt_v[pl.ds(lev * LANES, LANES)]

            def m_body(m, mcarry):
                msplat = jnp.full((LANES,), m, jnp.int32)
                mug = plsc.load_gather(mu_v, [msplat, idxv])
                alg = plsc.load_gather(al_v, [msplat, idxv])
                beg = plsc.load_gather(be_v, [msplat, idxv])
                res = mug + (alg - mug) * jnp.exp(-beg * dtv)
                out_v[m, pl.ds(lev * LANES, LANES)] = res
                return mcarry

            lax.fori_loop(0, M, m_body, 0)
            return qcarry

        lax.fori_loop(0, QV, q_body, 0)

        pltpu.async_copy(out_v, out_hbm.at[row_v], sem3).wait()
        return carry

    lax.fori_loop(0, NPATH // 32, path_body, 0)


@jax.jit
def kernel(event_times, mu, alpha, beta, query_times):
    ev2 = event_times.reshape(NPATH, L)
    q2 = query_times.reshape(NPATH, LE)
    mu2 = mu.reshape(B * M * P, L)
    al2 = alpha.reshape(B * M * P, L)
    be2 = beta.reshape(B * M * P, L)

    mesh = plsc.VectorSubcoreMesh(core_axis_name="c", subcore_axis_name="s")
    out = pl.kernel(
        _sc_body,
        mesh=mesh,
        out_type=jax.ShapeDtypeStruct((B * M * P, LE), jnp.float32),
        scratch_types=[
            pltpu.VMEM((L,), jnp.float32),
            pltpu.VMEM((LE,), jnp.float32),
            pltpu.VMEM((LE,), jnp.int32),
            pltpu.VMEM((LE,), jnp.float32),
            pltpu.VMEM((M,), jnp.int32),
            pltpu.VMEM((M, L), jnp.float32),
            pltpu.VMEM((M, L), jnp.float32),
            pltpu.VMEM((M, L), jnp.float32),
            pltpu.VMEM((M, LE), jnp.float32),
            pltpu.SemaphoreType.DMA,
            pltpu.SemaphoreType.DMA,
            pltpu.SemaphoreType.DMA,
            pltpu.SemaphoreType.DMA,
        ],
    )(ev2, q2, mu2, al2, be2)
    return out.reshape(B, M, P, LE)


# speedup vs baseline: 2544.2523x; 2544.2523x over previous
import jax
import jax.numpy as jnp
from jax import lax
from jax.experimental import pallas as pl
from jax.experimental.pallas import tpu as pltpu, tpu_sc as plsc

B, P, L, M, LE = 32, 16, 256, 64, 512
NPATH = B * P
LANES = 16
QV = LE // LANES


def _sc_body(ev_hbm, q_hbm, mu_hbm, al_hbm, be_hbm, out_hbm,
             ev_v, q_v, idx_v, dt_v, row_v, mu_v, al_v, be_v, out_v,
             sem0, sem1, sem2, sem3):
    nc = 2
    wid = lax.axis_index("s") * nc + lax.axis_index("c")
    iota = lax.iota(jnp.int32, LANES)

    def path_body(i, carry):
        path = wid * 16 + i
        b = path // P
        p = path - b * P
        base = b * (M * P) + p

        pltpu.sync_copy(ev_hbm.at[path], ev_v)
        pltpu.sync_copy(q_hbm.at[path], q_v)

        for k in range(M // LANES):
            row_v[pl.ds(k * LANES, LANES)] = base + (k * LANES + iota) * P

        cp0 = pltpu.async_copy(mu_hbm.at[row_v], mu_v, sem0)
        cp1 = pltpu.async_copy(al_hbm.at[row_v], al_v, sem1)
        cp2 = pltpu.async_copy(be_hbm.at[row_v], be_v, sem2)

        def bs_body(lev, bcarry):
            q = q_v[pl.ds(lev * LANES, LANES)]
            lo = jnp.zeros((LANES,), jnp.int32)
            hi = jnp.full((LANES,), L, jnp.int32)
            for _step in range(8):
                mid = lax.shift_right_logical(lo + hi, 1)
                ev_mid = plsc.load_gather(ev_v, [mid])
                pred = ev_mid < q
                lo = jnp.where(pred, mid + 1, lo)
                hi = jnp.where(pred, hi, mid)
            last = lo - 1
            clamped = jnp.maximum(last, 0)
            t_last = plsc.load_gather(ev_v, [clamped])
            t_last = jnp.where(last < 0, jnp.zeros_like(t_last), t_last)
            idx_v[pl.ds(lev * LANES, LANES)] = clamped
            dt_v[pl.ds(lev * LANES, LANES)] = q - t_last
            return bcarry

        lax.fori_loop(0, QV, bs_body, 0)

        cp0.wait()
        cp1.wait()
        cp2.wait()

        def q_body(lev, qcarry):
            idxv = idx_v[pl.ds(lev * LANES, LANES)]
            dtv = dt_v[pl.ds(lev * LANES, LANES)]

            def m_body(m, mcarry):
                msplat = jnp.full((LANES,), m, jnp.int32)
                mug = plsc.load_gather(mu_v, [msplat, idxv])
                alg = plsc.load_gather(al_v, [msplat, idxv])
                beg = plsc.load_gather(be_v, [msplat, idxv])
                res = mug + (alg - mug) * jnp.exp(-beg * dtv)
                out_v[m, pl.ds(lev * LANES, LANES)] = res
                return mcarry

            lax.fori_loop(0, M, m_body, 0)
            return qcarry

        lax.fori_loop(0, QV, q_body, 0)

        pltpu.async_copy(out_v, out_hbm.at[row_v], sem3).wait()
        return carry

    lax.fori_loop(0, NPATH // 32, path_body, 0)


@jax.jit
def kernel(event_times, mu, alpha, beta, query_times):
    ev2 = event_times.reshape(NPATH, L)
    q2 = query_times.reshape(NPATH, LE)
    mu2 = mu.reshape(B * M * P, L)
    al2 = alpha.reshape(B * M * P, L)
    be2 = beta.reshape(B * M * P, L)

    mesh = plsc.VectorSubcoreMesh(core_axis_name="c", subcore_axis_name="s")
    out = pl.kernel(
        _sc_body,
        mesh=mesh,
        compiler_params=pltpu.CompilerParams(needs_layout_passes=False),
        out_type=jax.ShapeDtypeStruct((B * M * P, LE), jnp.float32),
        scratch_types=[
            pltpu.VMEM((L,), jnp.float32),
            pltpu.VMEM((LE,), jnp.float32),
            pltpu.VMEM((LE,), jnp.int32),
            pltpu.VMEM((LE,), jnp.float32),
            pltpu.VMEM((M,), jnp.int32),
            pltpu.VMEM((M, L), jnp.float32),
            pltpu.VMEM((M, L), jnp.float32),
            pltpu.VMEM((M, L), jnp.float32),
            pltpu.VMEM((M, LE), jnp.float32),
            pltpu.SemaphoreType.DMA,
            pltpu.SemaphoreType.DMA,
            pltpu.SemaphoreType.DMA,
            pltpu.SemaphoreType.DMA,
        ],
    )(ev2, q2, mu2, al2, be2)
    return out.reshape(B, M, P, LE)


# speedup vs baseline: 6181.3108x; 2.4295x over previous
import jax
import jax.numpy as jnp
from jax import lax
from jax.experimental import pallas as pl
from jax.experimental.pallas import tpu as pltpu, tpu_sc as plsc

B, P, L, M, LE = 32, 16, 256, 64, 512
NPATH = B * P
LANES = 16
QV = LE // LANES


def _sc_body(ev_hbm, q_hbm, mu_hbm, al_hbm, be_hbm, out_hbm,
             ev_v, q_v, idx_v, dt_v, row_v, mu_v, al_v, be_v, out_v,
             sem0, sem1, sem2, sem3):
    nc = 2
    wid = lax.axis_index("s") * nc + lax.axis_index("c")
    iota = lax.iota(jnp.int32, LANES)

    def path_body(i, carry):
        path = wid * 16 + i
        b = path // P
        p = path - b * P
        base = b * (M * P) + p

        pltpu.sync_copy(ev_hbm.at[path], ev_v)
        pltpu.sync_copy(q_hbm.at[path], q_v)

        for k in range(M // LANES):
            row_v[pl.ds(k * LANES, LANES)] = base + (k * LANES + iota) * P

        cp0 = pltpu.async_copy(mu_hbm.at[row_v], mu_v, sem0)
        cp1 = pltpu.async_copy(al_hbm.at[row_v], al_v, sem1)
        cp2 = pltpu.async_copy(be_hbm.at[row_v], be_v, sem2)

        def bs_body(lev, bcarry):
            q = q_v[pl.ds(lev * LANES, LANES)]
            lo = jnp.zeros((LANES,), jnp.int32)
            hi = jnp.full((LANES,), L, jnp.int32)
            for _step in range(8):
                mid = lax.shift_right_logical(lo + hi, 1)
                ev_mid = plsc.load_gather(ev_v, [mid])
                pred = ev_mid < q
                lo = jnp.where(pred, mid + 1, lo)
                hi = jnp.where(pred, hi, mid)
            last = lo - 1
            clamped = jnp.maximum(last, 0)
            t_last = plsc.load_gather(ev_v, [clamped])
            t_last = jnp.where(last < 0, jnp.zeros_like(t_last), t_last)
            idx_v[pl.ds(lev * LANES, LANES)] = clamped
            dt_v[pl.ds(lev * LANES, LANES)] = t_last - q
            return bcarry

        lax.fori_loop(0, QV, bs_body, 0)

        cp0.wait()
        cp1.wait()
        cp2.wait()

        def q_body(lev, qcarry):
            idxv = idx_v[pl.ds(lev * LANES, LANES)]
            dtn = dt_v[pl.ds(lev * LANES, LANES)]
            obase = lev * LANES

            @plsc.parallel_loop(0, M, 1, unroll=4)
            def m_body(m):
                msplat = jnp.full((LANES,), m, jnp.int32)
                mug = plsc.load_gather(mu_v, [msplat, idxv])
                alg = plsc.load_gather(al_v, [msplat, idxv])
                beg = plsc.load_gather(be_v, [msplat, idxv])
                res = mug + (alg - mug) * jnp.exp(beg * dtn)
                out_v[m, pl.ds(obase, LANES)] = res

            return qcarry

        lax.fori_loop(0, QV, q_body, 0)

        pltpu.async_copy(out_v, out_hbm.at[row_v], sem3).wait()
        return carry

    lax.fori_loop(0, NPATH // 32, path_body, 0)


@jax.jit
def kernel(event_times, mu, alpha, beta, query_times):
    ev2 = event_times.reshape(NPATH, L)
    q2 = query_times.reshape(NPATH, LE)
    mu2 = mu.reshape(B * M * P, L)
    al2 = alpha.reshape(B * M * P, L)
    be2 = beta.reshape(B * M * P, L)

    mesh = plsc.VectorSubcoreMesh(core_axis_name="c", subcore_axis_name="s")
    out = pl.kernel(
        _sc_body,
        mesh=mesh,
        compiler_params=pltpu.CompilerParams(needs_layout_passes=False),
        out_type=jax.ShapeDtypeStruct((B * M * P, LE), jnp.float32),
        scratch_types=[
            pltpu.VMEM((L,), jnp.float32),
            pltpu.VMEM((LE,), jnp.float32),
            pltpu.VMEM((LE,), jnp.int32),
            pltpu.VMEM((LE,), jnp.float32),
            pltpu.VMEM((M,), jnp.int32),
            pltpu.VMEM((M, L), jnp.float32),
            pltpu.VMEM((M, L), jnp.float32),
            pltpu.VMEM((M, L), jnp.float32),
            pltpu.VMEM((M, LE), jnp.float32),
            pltpu.SemaphoreType.DMA,
            pltpu.SemaphoreType.DMA,
            pltpu.SemaphoreType.DMA,
            pltpu.SemaphoreType.DMA,
        ],
    )(ev2, q2, mu2, al2, be2)
    return out.reshape(B, M, P, LE)


# speedup vs baseline: 7710.9299x; 1.2475x over previous
import jax
import jax.numpy as jnp
from jax import lax
from jax.experimental import pallas as pl
from jax.experimental.pallas import tpu as pltpu, tpu_sc as plsc

B, P, L, M, LE = 32, 16, 256, 64, 512
NPATH = B * P
LANES = 16
QV = LE // LANES
MH = M // 2


def _sc_body(ev_hbm, q_hbm, mu_hbm, al_hbm, be_hbm, out_hbm,
             ev0_v, ev1_v, q0_v, q1_v, idx_v, dt_v,
             rowA_v, rowB_v, rowO0_v, rowO1_v,
             muA_v, alA_v, beA_v, muB_v, alB_v, beB_v,
             out0_v, out1_v,
             semE, semA, semB, semO0, semO1):
    nc = 2
    wid = lax.axis_index("s") * nc + lax.axis_index("c")
    iota = lax.iota(jnp.int32, LANES)
    p0 = wid * 16

    def set_rows(row_ref, base, half):
        for k in range(MH // LANES):
            row_ref[pl.ds(k * LANES, LANES)] = (
                base + (half * MH + k * LANES + iota) * P)

    def fire_evq(path, ev_ref, q_ref):
        pltpu.async_copy(ev_hbm.at[path], ev_ref, semE)
        pltpu.async_copy(q_hbm.at[path], q_ref, semE)

    def wait_evq(ev_ref, q_ref):
        pltpu.make_async_copy(ev_hbm.at[0], ev_ref, semE).wait()
        pltpu.make_async_copy(q_hbm.at[0], q_ref, semE).wait()

    def fire_params(base, half, row_ref, mu_ref, al_ref, be_ref, sem):
        set_rows(row_ref, base, half)
        pltpu.async_copy(mu_hbm.at[row_ref], mu_ref, sem)
        pltpu.async_copy(al_hbm.at[row_ref], al_ref, sem)
        pltpu.async_copy(be_hbm.at[row_ref], be_ref, sem)

    def wait_params(row_ref, mu_ref, al_ref, be_ref, sem):
        pltpu.make_async_copy(mu_hbm.at[row_ref], mu_ref, sem).wait()
        pltpu.make_async_copy(al_hbm.at[row_ref], al_ref, sem).wait()
        pltpu.make_async_copy(be_hbm.at[row_ref], be_ref, sem).wait()

    def search(ev_ref, q_ref):
        @plsc.parallel_loop(0, QV, 1, unroll=4)
        def bs_body(lev):
            q = q_ref[pl.ds(lev * LANES, LANES)]
            lo = jnp.zeros((LANES,), jnp.int32)
            hi = jnp.full((LANES,), L, jnp.int32)
            for _ in range(8):
                mid = lax.shift_right_logical(lo + hi, 1)
                pred = plsc.load_gather(ev_ref, [mid]) < q
                lo = jnp.where(pred, mid + 1, lo)
                hi = jnp.where(pred, hi, mid)
            last = lo - 1
            clamped = jnp.maximum(last, 0)
            t_last = plsc.load_gather(ev_ref, [clamped])
            t_last = jnp.where(last < 0, jnp.zeros_like(t_last), t_last)
            idx_v[pl.ds(lev * LANES, LANES)] = clamped
            dt_v[pl.ds(lev * LANES, LANES)] = t_last - q

    def compute_half(mu_ref, al_ref, be_ref, out_ref):
        def q_body(lev, qcarry):
            idxv = idx_v[pl.ds(lev * LANES, LANES)]
            dtn = dt_v[pl.ds(lev * LANES, LANES)]
            obase = lev * LANES

            @plsc.parallel_loop(0, MH, 1, unroll=4)
            def m_body(m):
                msplat = jnp.full((LANES,), m, jnp.int32)
                mug = plsc.load_gather(mu_ref, [msplat, idxv])
                alg = plsc.load_gather(al_ref, [msplat, idxv])
                beg = plsc.load_gather(be_ref, [msplat, idxv])
                res = mug + (alg - mug) * jnp.exp(beg * dtn)
                out_ref[m, pl.ds(obase, LANES)] = res

            return qcarry

        lax.fori_loop(0, QV, q_body, 0)

    def fire_scatter(base, half, row_ref, out_ref, sem):
        set_rows(row_ref, base, half)
        pltpu.async_copy(out_ref, out_hbm.at[row_ref], sem)

    def wait_scatter(row_ref, out_ref, sem):
        pltpu.make_async_copy(out_ref, out_hbm.at[row_ref], sem).wait()

    evq = ((ev0_v, q0_v), (ev1_v, q1_v))
    pA = (rowA_v, muA_v, alA_v, beA_v, semA)
    pB = (rowB_v, muB_v, alB_v, beB_v, semB)
    o0 = (rowO0_v, out0_v, semO0)
    o1 = (rowO1_v, out1_v, semO1)

    fire_evq(p0, *evq[0])
    fire_params(p0 // P * (M * P) + p0 % P, 0, *pA)

    def pair_body(h, carry):
        for j in range(2):
            path = p0 + 2 * h + j
            base = (path // P) * (M * P) + path % P
            nbase = ((path + 1) // P) * (M * P) + (path + 1) % P

            wait_evq(*evq[j])
            search(*evq[j])
            wait_params(*pA)
            fire_params(base, 1, *pB)
            if j == 0:
                fire_evq(path + 1, *evq[1])

            if j == 0:
                @pl.when(h > 0)
                def _():
                    wait_scatter(*o0)
            else:
                wait_scatter(*o0)
            compute_half(pA[1], pA[2], pA[3], o0[1])
            fire_scatter(base, 0, *o0)

            wait_params(*pB)
            if j == 0:
                fire_params(nbase, 0, *pA)
            else:
                @pl.when(h < 7)
                def _():
                    fire_params(nbase, 0, *pA)
                    fire_evq(path + 1, *evq[0])

            if j == 0:
                @pl.when(h > 0)
                def _():
                    wait_scatter(*o1)
            else:
                wait_scatter(*o1)
            compute_half(pB[1], pB[2], pB[3], o1[1])
            fire_scatter(base, 1, *o1)
        return carry

    lax.fori_loop(0, 8, pair_body, 0)

    wait_scatter(*o0)
    wait_scatter(*o1)


@jax.jit
def kernel(event_times, mu, alpha, beta, query_times):
    ev2 = event_times.reshape(NPATH, L)
    q2 = query_times.reshape(NPATH, LE)
    mu2 = mu.reshape(B * M * P, L)
    al2 = alpha.reshape(B * M * P, L)
    be2 = beta.reshape(B * M * P, L)

    mesh = plsc.VectorSubcoreMesh(core_axis_name="c", subcore_axis_name="s")
    out = pl.kernel(
        _sc_body,
        mesh=mesh,
        compiler_params=pltpu.CompilerParams(needs_layout_passes=False),
        out_type=jax.ShapeDtypeStruct((B * M * P, LE), jnp.float32),
        scratch_types=[
            pltpu.VMEM((L,), jnp.float32),
            pltpu.VMEM((L,), jnp.float32),
            pltpu.VMEM((LE,), jnp.float32),
            pltpu.VMEM((LE,), jnp.float32),
            pltpu.VMEM((LE,), jnp.int32),
            pltpu.VMEM((LE,), jnp.float32),
            pltpu.VMEM((MH,), jnp.int32),
            pltpu.VMEM((MH,), jnp.int32),
            pltpu.VMEM((MH,), jnp.int32),
            pltpu.VMEM((MH,), jnp.int32),
            pltpu.VMEM((MH, L), jnp.float32),
            pltpu.VMEM((MH, L), jnp.float32),
            pltpu.VMEM((MH, L), jnp.float32),
            pltpu.VMEM((MH, L), jnp.float32),
            pltpu.VMEM((MH, L), jnp.float32),
            pltpu.VMEM((MH, L), jnp.float32),
            pltpu.VMEM((MH, LE), jnp.float32),
            pltpu.VMEM((MH, LE), jnp.float32),
            pltpu.SemaphoreType.DMA,
            pltpu.SemaphoreType.DMA,
            pltpu.SemaphoreType.DMA,
            pltpu.SemaphoreType.DMA,
            pltpu.SemaphoreType.DMA,
        ],
    )(ev2, q2, mu2, al2, be2)
    return out.reshape(B, M, P, LE)


# speedup vs baseline: 8184.0342x; 1.0614x over previous
import jax
import jax.numpy as jnp
from jax import lax
from jax.experimental import pallas as pl
from jax.experimental.pallas import tpu as pltpu, tpu_sc as plsc

B, P, L, M, LE = 32, 16, 256, 64, 512
NPATH = B * P
LANES = 16
QV = LE // LANES
MH = M // 2


def _sc_body(ev_hbm, q_hbm, mu_hbm, al_hbm, be_hbm, out_hbm,
             ev0_v, ev1_v, q0_v, q1_v, idx_v, dt_v,
             rowA_v, rowB_v, rowO0_v, rowO1_v,
             muA_v, alA_v, beA_v, muB_v, alB_v, beB_v,
             out0_v, out1_v,
             semE, semA, semB, semO0, semO1):
    nc = 2
    wid = lax.axis_index("s") * nc + lax.axis_index("c")
    iota = lax.iota(jnp.int32, LANES)
    p0 = wid * 16

    def set_rows(row_ref, base, half):
        for k in range(MH // LANES):
            row_ref[pl.ds(k * LANES, LANES)] = (
                base + (half * MH + k * LANES + iota) * P)

    def fire_evq(path, ev_ref, q_ref):
        pltpu.async_copy(ev_hbm.at[path], ev_ref, semE)
        pltpu.async_copy(q_hbm.at[path], q_ref, semE)

    def wait_evq(ev_ref, q_ref):
        pltpu.make_async_copy(ev_hbm.at[0], ev_ref, semE).wait()
        pltpu.make_async_copy(q_hbm.at[0], q_ref, semE).wait()

    def fire_params(base, half, row_ref, mu_ref, al_ref, be_ref, sem):
        set_rows(row_ref, base, half)
        pltpu.async_copy(mu_hbm.at[row_ref], mu_ref, sem)
        pltpu.async_copy(al_hbm.at[row_ref], al_ref, sem)
        pltpu.async_copy(be_hbm.at[row_ref], be_ref, sem)

    def wait_params(row_ref, mu_ref, al_ref, be_ref, sem):
        pltpu.make_async_copy(mu_hbm.at[row_ref], mu_ref, sem).wait()
        pltpu.make_async_copy(al_hbm.at[row_ref], al_ref, sem).wait()
        pltpu.make_async_copy(be_hbm.at[row_ref], be_ref, sem).wait()

    def search(ev_ref, q_ref):
        @plsc.parallel_loop(0, QV, 1, unroll=8)
        def bs_body(lev):
            q = q_ref[pl.ds(lev * LANES, LANES)]
            lo = jnp.zeros((LANES,), jnp.int32)
            hi = jnp.full((LANES,), L, jnp.int32)
            for _ in range(8):
                mid = lax.shift_right_logical(lo + hi, 1)
                pred = plsc.load_gather(ev_ref, [mid]) < q
                lo = jnp.where(pred, mid + 1, lo)
                hi = jnp.where(pred, hi, mid)
            last = lo - 1
            clamped = jnp.maximum(last, 0)
            t_last = plsc.load_gather(ev_ref, [clamped])
            t_last = jnp.where(last < 0, jnp.zeros_like(t_last), t_last)
            idx_v[pl.ds(lev * LANES, LANES)] = clamped
            dt_v[pl.ds(lev * LANES, LANES)] = t_last - q

    def compute_half(mu_ref, al_ref, be_ref, out_ref):
        def q_body(lev, qcarry):
            idxv = idx_v[pl.ds(lev * LANES, LANES)]
            dtn = dt_v[pl.ds(lev * LANES, LANES)]
            obase = lev * LANES

            @plsc.parallel_loop(0, MH, 1, unroll=8)
            def m_body(m):
                msplat = jnp.full((LANES,), m, jnp.int32)
                mug = plsc.load_gather(mu_ref, [msplat, idxv])
                alg = plsc.load_gather(al_ref, [msplat, idxv])
                beg = plsc.load_gather(be_ref, [msplat, idxv])
                res = mug + (alg - mug) * jnp.exp(beg * dtn)
                out_ref[m, pl.ds(obase, LANES)] = res

            return qcarry

        lax.fori_loop(0, QV, q_body, 0)

    def fire_scatter(base, half, row_ref, out_ref, sem):
        set_rows(row_ref, base, half)
        pltpu.async_copy(out_ref, out_hbm.at[row_ref], sem)

    def wait_scatter(row_ref, out_ref, sem):
        pltpu.make_async_copy(out_ref, out_hbm.at[row_ref], sem).wait()

    evq = ((ev0_v, q0_v), (ev1_v, q1_v))
    pA = (rowA_v, muA_v, alA_v, beA_v, semA)
    pB = (rowB_v, muB_v, alB_v, beB_v, semB)
    o0 = (rowO0_v, out0_v, semO0)
    o1 = (rowO1_v, out1_v, semO1)

    fire_evq(p0, *evq[0])
    fire_params(p0 // P * (M * P) + p0 % P, 0, *pA)

    def pair_body(h, carry):
        for j in range(2):
            path = p0 + 2 * h + j
            base = (path // P) * (M * P) + path % P
            nbase = ((path + 1) // P) * (M * P) + (path + 1) % P

            wait_evq(*evq[j])
            search(*evq[j])
            wait_params(*pA)
            fire_params(base, 1, *pB)
            if j == 0:
                fire_evq(path + 1, *evq[1])

            if j == 0:
                @pl.when(h > 0)
                def _():
                    wait_scatter(*o0)
            else:
                wait_scatter(*o0)
            compute_half(pA[1], pA[2], pA[3], o0[1])
            fire_scatter(base, 0, *o0)

            wait_params(*pB)
            if j == 0:
                fire_params(nbase, 0, *pA)
            else:
                @pl.when(h < 7)
                def _():
                    fire_params(nbase, 0, *pA)
                    fire_evq(path + 1, *evq[0])

            if j == 0:
                @pl.when(h > 0)
                def _():
                    wait_scatter(*o1)
            else:
                wait_scatter(*o1)
            compute_half(pB[1], pB[2], pB[3], o1[1])
            fire_scatter(base, 1, *o1)
        return carry

    lax.fori_loop(0, 8, pair_body, 0)

    wait_scatter(*o0)
    wait_scatter(*o1)


@jax.jit
def kernel(event_times, mu, alpha, beta, query_times):
    ev2 = event_times.reshape(NPATH, L)
    q2 = query_times.reshape(NPATH, LE)
    mu2 = mu.reshape(B * M * P, L)
    al2 = alpha.reshape(B * M * P, L)
    be2 = beta.reshape(B * M * P, L)

    mesh = plsc.VectorSubcoreMesh(core_axis_name="c", subcore_axis_name="s")
    out = pl.kernel(
        _sc_body,
        mesh=mesh,
        compiler_params=pltpu.CompilerParams(needs_layout_passes=False),
        out_type=jax.ShapeDtypeStruct((B * M * P, LE), jnp.float32),
        scratch_types=[
            pltpu.VMEM((L,), jnp.float32),
            pltpu.VMEM((L,), jnp.float32),
            pltpu.VMEM((LE,), jnp.float32),
            pltpu.VMEM((LE,), jnp.float32),
            pltpu.VMEM((LE,), jnp.int32),
            pltpu.VMEM((LE,), jnp.float32),
            pltpu.VMEM((MH,), jnp.int32),
            pltpu.VMEM((MH,), jnp.int32),
            pltpu.VMEM((MH,), jnp.int32),
            pltpu.VMEM((MH,), jnp.int32),
            pltpu.VMEM((MH, L), jnp.float32),
            pltpu.VMEM((MH, L), jnp.float32),
            pltpu.VMEM((MH, L), jnp.float32),
            pltpu.VMEM((MH, L), jnp.float32),
            pltpu.VMEM((MH, L), jnp.float32),
            pltpu.VMEM((MH, L), jnp.float32),
            pltpu.VMEM((MH, LE), jnp.float32),
            pltpu.VMEM((MH, LE), jnp.float32),
            pltpu.SemaphoreType.DMA,
            pltpu.SemaphoreType.DMA,
            pltpu.SemaphoreType.DMA,
            pltpu.SemaphoreType.DMA,
            pltpu.SemaphoreType.DMA,
        ],
    )(ev2, q2, mu2, al2, be2)
    return out.reshape(B, M, P, LE)
